# baseline (speedup 1.0000x reference)
import functools

import jax
import jax.numpy as jnp
from jax import lax
from jax.experimental import pallas as pl
from jax.experimental.pallas import tpu as pltpu
from jax.experimental.pallas import tpu_sc as plsc

BATCH = 16384
D = 32
SUP = 128
NC = 2
NS = 16
NW = NC * NS
BPW = BATCH // NW
CHUNK = 128
NCHUNK = BPW // CHUNK

_sc_mesh = plsc.VectorSubcoreMesh(core_axis_name="c", subcore_axis_name="s")


@functools.partial(
    pl.kernel,
    mesh=_sc_mesh,
    out_type=(
        jax.ShapeDtypeStruct((BATCH, SUP), jnp.float32),
        jax.ShapeDtypeStruct((BATCH, SUP), jnp.float32),
    ),
    scratch_types=[
        pltpu.VMEM((NCHUNK, CHUNK), jnp.int32),
        pltpu.VMEM((NCHUNK, CHUNK), jnp.int32),
        pltpu.VMEM((2, CHUNK, SUP), jnp.float32),
        pltpu.VMEM((2, CHUNK, SUP), jnp.float32),
        pltpu.SemaphoreType.DMA,
        pltpu.SemaphoreType.DMA,
        pltpu.SemaphoreType.DMA,
        pltpu.SemaphoreType.DMA,
    ],
)
def _sc_gather(uid_hbm, iid_hbm, ut_hbm, it_hbm, ue_hbm, ie_hbm,
               uidx_v, iidx_v, ubuf_v, ibuf_v, su0, su1, si0, si1):
    wid = lax.axis_index("s") * NC + lax.axis_index("c")
    base = wid * BPW
    sems_u = (su0, su1)
    sems_i = (si0, si1)
    pltpu.sync_copy(uid_hbm.at[wid], uidx_v)
    pltpu.sync_copy(iid_hbm.at[wid], iidx_v)
    cps = [None] * (2 * NCHUNK)
    cps[0] = pltpu.async_copy(ut_hbm.at[uidx_v.at[0]], ubuf_v.at[0], sems_u[0])
    cps[1] = pltpu.async_copy(it_hbm.at[iidx_v.at[0]], ibuf_v.at[0], sems_i[0])
    for c in range(NCHUNK):
        s, n = c & 1, (c + 1) & 1
        if c + 1 < NCHUNK:
            cps[2 * (c + 1)] = pltpu.async_copy(
                ut_hbm.at[uidx_v.at[c + 1]], ubuf_v.at[n], sems_u[n])
            cps[2 * (c + 1) + 1] = pltpu.async_copy(
                it_hbm.at[iidx_v.at[c + 1]], ibuf_v.at[n], sems_i[n])
        cps[2 * c].wait()
        cps[2 * c + 1].wait()
        pltpu.sync_copy(ubuf_v.at[s], ue_hbm.at[pl.ds(base + c * CHUNK, CHUNK)])
        pltpu.sync_copy(ibuf_v.at[s], ie_hbm.at[pl.ds(base + c * CHUNK, CHUNK)])


BB = 2048
NPAD = 128


def _select_subrow(raw, off):
    x = raw[:, 0:D]
    for c in range(1, 4):
        x = jnp.where(off == c, raw[:, c * D:(c + 1) * D], x)
    return x


def _mlp_body(ue_ref, ie_ref, uoff_ref, ioff_ref, w1u_ref, w1i_ref, b1_ref,
              w2_ref, b2_ref, out_ref):
    xu = _select_subrow(ue_ref[...], uoff_ref[...])
    xi = _select_subrow(ie_ref[...], ioff_ref[...])
    h = jnp.dot(xu, w1u_ref[...], preferred_element_type=jnp.float32)
    h = h + jnp.dot(xi, w1i_ref[...], preferred_element_type=jnp.float32)
    h = jnp.maximum(h + b1_ref[...], 0.0)
    logits = jnp.dot(h, w2_ref[...], preferred_element_type=jnp.float32) + b2_ref[...]
    lane = lax.broadcasted_iota(jnp.int32, logits.shape, 1)
    masked = jnp.where(lane < 5, logits, -jnp.inf)
    m = jnp.max(masked, axis=1, keepdims=True)
    e = jnp.exp(masked - m)
    s = jnp.sum(e, axis=1, keepdims=True)
    out_ref[...] = (e / s)[:, :5]


def _mlp(ue, ie, uoff, ioff, w1u, w1i, b1, w2p, b2p):
    grid = (BATCH // BB,)
    return pl.pallas_call(
        _mlp_body,
        grid=grid,
        in_specs=[
            pl.BlockSpec((BB, SUP), lambda i: (i, 0)),
            pl.BlockSpec((BB, SUP), lambda i: (i, 0)),
            pl.BlockSpec((BB, 1), lambda i: (i, 0)),
            pl.BlockSpec((BB, 1), lambda i: (i, 0)),
            pl.BlockSpec((D, 64), lambda i: (0, 0)),
            pl.BlockSpec((D, 64), lambda i: (0, 0)),
            pl.BlockSpec((1, 64), lambda i: (0, 0)),
            pl.BlockSpec((64, NPAD), lambda i: (0, 0)),
            pl.BlockSpec((1, NPAD), lambda i: (0, 0)),
        ],
        out_specs=pl.BlockSpec((BB, 5), lambda i: (i, 0)),
        out_shape=jax.ShapeDtypeStruct((BATCH, 5), jnp.float32),
    )(ue, ie, uoff, ioff, w1u, w1i, b1, w2p, b2p)


def kernel(user_ids, item_ids, user_table, item_table, W1, b1, W2, b2):
    uid = user_ids.astype(jnp.int32)
    iid = item_ids.astype(jnp.int32)
    usup = jnp.reshape(uid >> 2, (NW, NCHUNK, CHUNK))
    isup = jnp.reshape(iid >> 2, (NW, NCHUNK, CHUNK))
    ut128 = jnp.reshape(user_table, (user_table.shape[0] // 4, SUP))
    it128 = jnp.reshape(item_table, (item_table.shape[0] // 4, SUP))
    ue, ie = _sc_gather(usup, isup, ut128, it128)
    uoff = jnp.reshape(uid & 3, (BATCH, 1))
    ioff = jnp.reshape(iid & 3, (BATCH, 1))
    w1u = jnp.transpose(W1[:, :D])
    w1i = jnp.transpose(W1[:, D:])
    w2p = jnp.pad(jnp.transpose(W2), ((0, 0), (0, NPAD - 5)))
    b2p = jnp.pad(jnp.reshape(b2, (1, 5)), ((0, 0), (0, NPAD - 5)))
    return _mlp(ue, ie, uoff, ioff, w1u, w1i, jnp.reshape(b1, (1, 64)), w2p, b2p)
